# baseline (speedup 1.0000x reference)
import functools

import jax
import jax.numpy as jnp
from jax import lax
from jax.experimental import pallas as pl
from jax.experimental.pallas import tpu as pltpu
from jax.experimental.pallas import tpu_sc as plsc


_CC = 2048
_WINDOW = 8


def kernel(edge_index, W):
    E = edge_index.shape[1]
    D = W.shape[1]
    info = plsc.get_sparse_core_info()
    nw = info.num_cores * info.num_subcores
    n_tile_rows = D // 8
    sc_per_row = nw // n_tile_rows
    n_chunks = -(-E // _CC)
    per_sc = -(-n_chunks // sc_per_row)
    last_start = E - _CC
    mesh = plsc.VectorSubcoreMesh(core_axis_name="c", subcore_axis_name="s")

    @functools.partial(
        pl.kernel,
        mesh=mesh,
        out_type=jax.ShapeDtypeStruct((D, E), jnp.float32),
        scratch_types=[
            pltpu.MemorySpace.VMEM((16,), jnp.float32),
            pltpu.MemorySpace.VMEM((8, _CC), jnp.float32),
            pltpu.SemaphoreType.DMA,
        ],
    )
    def fill_kernel(w_hbm, o_hbm, wv, buf, sem):
        wid = lax.axis_index("s") * info.num_cores + lax.axis_index("c")
        tile_row = wid // sc_per_row
        quarter = wid % sc_per_row
        for s in range(8):
            row = tile_row * 8 + s
            pltpu.sync_copy(w_hbm.at[pl.ds(row * 16, 16)], wv)
            splat = wv[...]

            @pl.loop(0, _CC, step=16)
            def _fill(i):
                buf[s, pl.ds(i, 16)] = splat

        lo = quarter * per_sc
        hi = jnp.minimum(lo + per_sc, n_chunks)

        def _dst(j):
            col0 = jnp.minimum(j * _CC, last_start)
            return o_hbm.at[pl.ds(tile_row * 8, 8), pl.ds(col0, _CC)]

        @pl.loop(lo, hi)
        def _fire(j):
            pltpu.make_async_copy(buf, _dst(j), sem).start()

            @pl.when(j >= lo + _WINDOW)
            def _():
                pltpu.make_async_copy(buf, _dst(j - _WINDOW), sem).wait()

        @pl.loop(jnp.maximum(hi - _WINDOW, lo), hi)
        def _drain(j):
            pltpu.make_async_copy(buf, _dst(j), sem).wait()

    w_rep = jnp.repeat(W.reshape(D), 16)
    out_t = fill_kernel(w_rep)
    return out_t.T
